# baseline (speedup 1.0000x reference)
import functools

import jax
import jax.numpy as jnp
from jax import lax
from jax.experimental import pallas as pl
from jax.experimental.pallas import tpu as pltpu
from jax.experimental.pallas import tpu_sc as plsc

WORD_DIM = 64
LANE = 16
IDX_MINOR = 128
NUM_CORES = 2
NUM_SUBCORES = 16
NUM_WORKERS = NUM_CORES * NUM_SUBCORES


def _transpose_block(g_ref, gt_ref, b, row_vecs):
    def dg_step(dg, carry):
        for dd in range(8):
            d = dg * 8 + dd
            col = jnp.full((LANE,), d, jnp.int32)
            for bcg in range(8):
                vals = plsc.load_gather(g_ref.at[b], [row_vecs[bcg], col])
                gt_ref[b, dg, dd, pl.ds(bcg * LANE, LANE)] = vals
        return carry

    lax.fori_loop(0, 8, dg_step, 0)


def _emb_body(rows_per_w, x_hbm, table_hbm, out_hbm, idx_v, g_v, gt_v,
              gsem0, gsem1, ssem0, ssem1):
    gsems = (gsem0, gsem1)
    ssems = (ssem0, ssem1)
    wid = lax.axis_index("s") * NUM_CORES + lax.axis_index("c")
    base = wid * rows_per_w

    lane = lax.iota(jnp.int32, LANE)
    row_vecs = [lane + bcg * LANE for bcg in range(8)]

    pltpu.sync_copy(x_hbm.at[pl.ds(base, rows_per_w)], idx_v)

    def fire_gather(b, i):
        pltpu.async_copy(table_hbm.at[idx_v.at[i]], g_v.at[b], gsems[b])

    def drain_gather(b):
        pltpu.make_async_copy(table_hbm.at[idx_v.at[0]], g_v.at[b],
                              gsems[b]).wait()

    def out_slice(i):
        blk = base + i
        sg = blk // 256
        bj = (blk // 8) % 32
        ss = blk % 8
        s = sg * 8 + ss
        return out_hbm.at[s, :, bj]

    def fire_store(b, i):
        pltpu.async_copy(gt_v.at[b], out_slice(i), ssems[b])

    def wait_store(b):
        pltpu.make_async_copy(gt_v.at[b], out_hbm.at[0, :, 0],
                              ssems[b]).wait()

    fire_gather(0, 0)
    n_pairs = rows_per_w // 2

    def pair_step(it, carry):
        i0 = 2 * it
        i1 = i0 + 1

        fire_gather(1, i1)
        drain_gather(0)

        @pl.when(it > 0)
        def _():
            wait_store(0)

        _transpose_block(g_v, gt_v, 0, row_vecs)
        fire_store(0, i0)

        @pl.when(it + 1 < n_pairs)
        def _():
            fire_gather(0, i0 + 2)

        drain_gather(1)

        @pl.when(it > 0)
        def _():
            wait_store(1)

        _transpose_block(g_v, gt_v, 1, row_vecs)
        fire_store(1, i1)
        return carry

    lax.fori_loop(0, n_pairs, pair_step, 0)
    wait_store(0)
    wait_store(1)


def kernel(x, emb_table):
    batch, seq = x.shape
    total = batch * seq
    n_rows = total // IDX_MINOR
    rows_per_w = n_rows // NUM_WORKERS
    assert n_rows % NUM_WORKERS == 0 and rows_per_w % 2 == 0
    n_btile = batch // IDX_MINOR
    n_stile = seq // 8

    xv = (x.astype(jnp.int32)
          .reshape(n_btile, IDX_MINOR, n_stile, 8)
          .transpose(2, 0, 3, 1)
          .reshape(n_rows, IDX_MINOR))

    mesh = plsc.VectorSubcoreMesh(core_axis_name="c", subcore_axis_name="s")
    emb_kernel = pl.kernel(
        functools.partial(_emb_body, rows_per_w),
        out_type=jax.ShapeDtypeStruct(
            (seq, WORD_DIM // 8, n_btile, 8, IDX_MINOR), emb_table.dtype),
        mesh=mesh,
        scratch_types=[
            pltpu.VMEM((rows_per_w, IDX_MINOR), jnp.int32),
            pltpu.VMEM((2, IDX_MINOR, WORD_DIM), emb_table.dtype),
            pltpu.VMEM((2, WORD_DIM // 8, 8, IDX_MINOR), emb_table.dtype),
            pltpu.SemaphoreType.DMA,
            pltpu.SemaphoreType.DMA,
            pltpu.SemaphoreType.DMA,
            pltpu.SemaphoreType.DMA,
        ],
        compiler_params=pltpu.CompilerParams(use_tc_tiling_on_sc=False,
                                             needs_layout_passes=False),
    )
    out5 = emb_kernel(xv, emb_table)
    return (out5.transpose(2, 4, 0, 1, 3)
            .reshape(batch, seq, WORD_DIM))


# speedup vs baseline: 1.1482x; 1.1482x over previous
import functools

import jax
import jax.numpy as jnp
from jax import lax
from jax.experimental import pallas as pl
from jax.experimental.pallas import tpu as pltpu
from jax.experimental.pallas import tpu_sc as plsc

WORD_DIM = 64
LANE = 16
IDX_MINOR = 128
NUM_CORES = 2
NUM_SUBCORES = 16
NUM_WORKERS = NUM_CORES * NUM_SUBCORES


def _transpose_block(g_ref, gt_ref, b, row_vecs):
    def dg_step(dg, carry):
        for dd in range(8):
            d = dg * 8 + dd
            col = jnp.full((LANE,), d, jnp.int32)
            vals = [plsc.load_gather(g_ref.at[b], [row_vecs[bcg], col])
                    for bcg in range(8)]
            for bcg in range(8):
                gt_ref[b, dg, dd, pl.ds(bcg * LANE, LANE)] = vals[bcg]
        return carry

    lax.fori_loop(0, 8, dg_step, 0)


def _emb_body(rows_per_w, xg_hbm, table_hbm, out_hbm,
              idx_v, g_v, gt_v, gsem0, gsem1, ssem0, ssem1):
    gsems = (gsem0, gsem1)
    ssems = (ssem0, ssem1)
    wid = lax.axis_index("s") * NUM_CORES + lax.axis_index("c")
    base = wid * rows_per_w

    lane = lax.iota(jnp.int32, LANE)
    row_vecs = [(lane + bcg * LANE) for bcg in range(8)]

    pltpu.sync_copy(xg_hbm.at[pl.ds(base, rows_per_w)], idx_v)

    def fire_gather(b, i):
        pltpu.async_copy(table_hbm.at[idx_v.at[i]], g_v.at[b], gsems[b])

    def drain_gather(b):
        pltpu.make_async_copy(table_hbm.at[idx_v.at[0]], g_v.at[b],
                              gsems[b]).wait()

    def out_slice(i):
        blk = base + i
        sg = blk // 256
        bj = (blk // 8) % 32
        ss = blk % 8
        s = sg * 8 + ss
        return out_hbm.at[s, :, bj]

    def fire_store(b, i):
        pltpu.async_copy(gt_v.at[b], out_slice(i), ssems[b])

    def wait_store(b):
        pltpu.make_async_copy(gt_v.at[b], out_hbm.at[0, :, 0],
                              ssems[b]).wait()

    fire_gather(0, 0)
    n_pairs = rows_per_w // 2

    def pair_step(it, carry):
        i0 = 2 * it
        i1 = i0 + 1

        fire_gather(1, i1)
        drain_gather(0)

        @pl.when(it > 0)
        def _():
            wait_store(0)

        _transpose_block(g_v, gt_v, 0, row_vecs)
        fire_store(0, i0)

        @pl.when(it + 1 < n_pairs)
        def _():
            fire_gather(0, i0 + 2)

        drain_gather(1)

        @pl.when(it > 0)
        def _():
            wait_store(1)

        _transpose_block(g_v, gt_v, 1, row_vecs)
        fire_store(1, i1)
        return carry

    lax.fori_loop(0, n_pairs, pair_step, 0)
    wait_store(0)
    wait_store(1)


def kernel(x, emb_table):
    batch, seq = x.shape
    total = batch * seq
    n_rows = total // IDX_MINOR
    rows_per_w = n_rows // NUM_WORKERS
    assert n_rows % NUM_WORKERS == 0 and rows_per_w % 2 == 0
    n_btile = batch // IDX_MINOR
    n_stile = seq // 8

    xv = (x.astype(jnp.int32)
          .reshape(n_btile, IDX_MINOR, n_stile, 8)
          .transpose(2, 0, 3, 1)
          .reshape(n_rows, IDX_MINOR))

    mesh = plsc.VectorSubcoreMesh(core_axis_name="c", subcore_axis_name="s")
    emb_kernel = pl.kernel(
        functools.partial(_emb_body, rows_per_w),
        out_type=jax.ShapeDtypeStruct(
            (seq, WORD_DIM // 8, n_btile, 8, IDX_MINOR), emb_table.dtype),
        mesh=mesh,
        scratch_types=[
            pltpu.VMEM((rows_per_w, IDX_MINOR), jnp.int32),
            pltpu.VMEM((2, IDX_MINOR, WORD_DIM), emb_table.dtype),
            pltpu.VMEM((2, WORD_DIM // 8, 8, IDX_MINOR), emb_table.dtype),
            pltpu.SemaphoreType.DMA,
            pltpu.SemaphoreType.DMA,
            pltpu.SemaphoreType.DMA,
            pltpu.SemaphoreType.DMA,
        ],
        compiler_params=pltpu.CompilerParams(use_tc_tiling_on_sc=False,
                                             needs_layout_passes=False),
    )
    out5 = emb_kernel(xv, emb_table)
    return (out5.transpose(2, 4, 0, 1, 3)
            .reshape(batch, seq, WORD_DIM))


# speedup vs baseline: 1.2149x; 1.0580x over previous
import functools

import jax
import jax.numpy as jnp
from jax import lax
from jax.experimental import pallas as pl
from jax.experimental.pallas import tpu as pltpu
from jax.experimental.pallas import tpu_sc as plsc

WORD_DIM = 64
LANE = 16
IDX_MINOR = 128
NUM_CORES = 2
NUM_SUBCORES = 16
NUM_WORKERS = NUM_CORES * NUM_SUBCORES


def _transpose_block(g_ref, gt_ref, b, row_vecs):
    @plsc.parallel_loop(0, WORD_DIM, unroll=2)
    def _(d):
        dg = d // 8
        dd = d % 8
        col = jnp.full((LANE,), d, jnp.int32)
        vals = [plsc.load_gather(g_ref.at[b], [row_vecs[bcg], col])
                for bcg in range(8)]
        for bcg in range(8):
            gt_ref[b, dg, dd, pl.ds(bcg * LANE, LANE)] = vals[bcg]


def _emb_body(rows_per_w, xg_hbm, table_hbm, out_hbm,
              idx_v, g_v, gt_v, gsem0, gsem1, ssem0, ssem1):
    gsems = (gsem0, gsem1)
    ssems = (ssem0, ssem1)
    wid = lax.axis_index("s") * NUM_CORES + lax.axis_index("c")
    base = wid * rows_per_w

    lane = lax.iota(jnp.int32, LANE)
    row_vecs = [(lane + bcg * LANE) for bcg in range(8)]

    pltpu.sync_copy(xg_hbm.at[pl.ds(base, rows_per_w)], idx_v)

    def fire_gather(b, i):
        pltpu.async_copy(table_hbm.at[idx_v.at[i]], g_v.at[b], gsems[b])

    def drain_gather(b):
        pltpu.make_async_copy(table_hbm.at[idx_v.at[0]], g_v.at[b],
                              gsems[b]).wait()

    def out_slice(i):
        blk = base + i
        sg = blk // 256
        bj = (blk // 8) % 32
        ss = blk % 8
        s = sg * 8 + ss
        return out_hbm.at[s, :, bj]

    def fire_store(b, i):
        pltpu.async_copy(gt_v.at[b], out_slice(i), ssems[b])

    def wait_store(b):
        pltpu.make_async_copy(gt_v.at[b], out_hbm.at[0, :, 0],
                              ssems[b]).wait()

    fire_gather(0, 0)
    n_pairs = rows_per_w // 2

    def pair_step(it, carry):
        i0 = 2 * it
        i1 = i0 + 1

        fire_gather(1, i1)
        drain_gather(0)

        @pl.when(it > 0)
        def _():
            wait_store(0)

        _transpose_block(g_v, gt_v, 0, row_vecs)
        fire_store(0, i0)

        @pl.when(it + 1 < n_pairs)
        def _():
            fire_gather(0, i0 + 2)

        drain_gather(1)

        @pl.when(it > 0)
        def _():
            wait_store(1)

        _transpose_block(g_v, gt_v, 1, row_vecs)
        fire_store(1, i1)
        return carry

    lax.fori_loop(0, n_pairs, pair_step, 0)
    wait_store(0)
    wait_store(1)


def kernel(x, emb_table):
    batch, seq = x.shape
    total = batch * seq
    n_rows = total // IDX_MINOR
    rows_per_w = n_rows // NUM_WORKERS
    assert n_rows % NUM_WORKERS == 0 and rows_per_w % 2 == 0
    n_btile = batch // IDX_MINOR
    n_stile = seq // 8

    xv = (x.astype(jnp.int32)
          .reshape(n_btile, IDX_MINOR, n_stile, 8)
          .transpose(2, 0, 3, 1)
          .reshape(n_rows, IDX_MINOR))

    mesh = plsc.VectorSubcoreMesh(core_axis_name="c", subcore_axis_name="s")
    emb_kernel = pl.kernel(
        functools.partial(_emb_body, rows_per_w),
        out_type=jax.ShapeDtypeStruct(
            (seq, WORD_DIM // 8, n_btile, 8, IDX_MINOR), emb_table.dtype),
        mesh=mesh,
        scratch_types=[
            pltpu.VMEM((rows_per_w, IDX_MINOR), jnp.int32),
            pltpu.VMEM((2, IDX_MINOR, WORD_DIM), emb_table.dtype),
            pltpu.VMEM((2, WORD_DIM // 8, 8, IDX_MINOR), emb_table.dtype),
            pltpu.SemaphoreType.DMA,
            pltpu.SemaphoreType.DMA,
            pltpu.SemaphoreType.DMA,
            pltpu.SemaphoreType.DMA,
        ],
        compiler_params=pltpu.CompilerParams(use_tc_tiling_on_sc=False,
                                             needs_layout_passes=False),
    )
    out5 = emb_kernel(xv, emb_table)
    return (out5.transpose(2, 4, 0, 1, 3)
            .reshape(batch, seq, WORD_DIM))


# speedup vs baseline: 2.2455x; 1.8483x over previous
import functools

import jax
import jax.numpy as jnp
from jax import lax
from jax.experimental import pallas as pl
from jax.experimental.pallas import tpu as pltpu
from jax.experimental.pallas import tpu_sc as plsc

WORD_DIM = 64
LANE = 16
IDX_MINOR = 128
NUM_CORES = 2
NUM_SUBCORES = 16
NUM_WORKERS = NUM_CORES * NUM_SUBCORES


def _transpose_block(g_ref, gt_ref, b, lane, rots):
    @plsc.parallel_loop(0, 32, unroll=2)
    def _(m):
        bc = m >> 2
        dc = m & 3
        rowv = bc * LANE + lane
        dbase = dc * LANE
        for k in range(16):
            colv = dbase + rots[k]
            val = plsc.load_gather(g_ref.at[b], [rowv, colv])
            plsc.store_scatter(gt_ref.at[b], [colv >> 3, colv & 7, rowv],
                               val)


def _emb_body(rows_per_w, xg_hbm, table_hbm, out_hbm,
              idx_v, g_v, gt_v, gsem0, gsem1, ssem0, ssem1):
    gsems = (gsem0, gsem1)
    ssems = (ssem0, ssem1)
    wid = lax.axis_index("s") * NUM_CORES + lax.axis_index("c")
    base = wid * rows_per_w

    lane = lax.iota(jnp.int32, LANE)
    rots = [(lane + k) & 15 for k in range(16)]

    pltpu.sync_copy(xg_hbm.at[pl.ds(base, rows_per_w)], idx_v)

    def fire_gather(b, i):
        pltpu.async_copy(table_hbm.at[idx_v.at[i]], g_v.at[b], gsems[b])

    def drain_gather(b):
        pltpu.make_async_copy(table_hbm.at[idx_v.at[0]], g_v.at[b],
                              gsems[b]).wait()

    def out_slice(i):
        blk = base + i
        sg = blk // 256
        bj = (blk // 8) % 32
        ss = blk % 8
        s = sg * 8 + ss
        return out_hbm.at[s, :, bj]

    def fire_store(b, i):
        pltpu.async_copy(gt_v.at[b], out_slice(i), ssems[b])

    def wait_store(b):
        pltpu.make_async_copy(gt_v.at[b], out_hbm.at[0, :, 0],
                              ssems[b]).wait()

    fire_gather(0, 0)
    n_pairs = rows_per_w // 2

    def pair_step(it, carry):
        i0 = 2 * it
        i1 = i0 + 1

        fire_gather(1, i1)
        drain_gather(0)

        @pl.when(it > 0)
        def _():
            wait_store(0)

        _transpose_block(g_v, gt_v, 0, lane, rots)
        fire_store(0, i0)

        @pl.when(it + 1 < n_pairs)
        def _():
            fire_gather(0, i0 + 2)

        drain_gather(1)

        @pl.when(it > 0)
        def _():
            wait_store(1)

        _transpose_block(g_v, gt_v, 1, lane, rots)
        fire_store(1, i1)
        return carry

    lax.fori_loop(0, n_pairs, pair_step, 0)
    wait_store(0)
    wait_store(1)


def kernel(x, emb_table):
    batch, seq = x.shape
    total = batch * seq
    n_rows = total // IDX_MINOR
    rows_per_w = n_rows // NUM_WORKERS
    assert n_rows % NUM_WORKERS == 0 and rows_per_w % 2 == 0
    n_btile = batch // IDX_MINOR
    n_stile = seq // 8

    xv = (x.astype(jnp.int32)
          .reshape(n_btile, IDX_MINOR, n_stile, 8)
          .transpose(2, 0, 3, 1)
          .reshape(n_rows, IDX_MINOR))

    mesh = plsc.VectorSubcoreMesh(core_axis_name="c", subcore_axis_name="s")
    emb_kernel = pl.kernel(
        functools.partial(_emb_body, rows_per_w),
        out_type=jax.ShapeDtypeStruct(
            (seq, WORD_DIM // 8, n_btile, 8, IDX_MINOR), emb_table.dtype),
        mesh=mesh,
        scratch_types=[
            pltpu.VMEM((rows_per_w, IDX_MINOR), jnp.int32),
            pltpu.VMEM((2, IDX_MINOR, WORD_DIM), emb_table.dtype),
            pltpu.VMEM((2, WORD_DIM // 8, 8, IDX_MINOR), emb_table.dtype),
            pltpu.SemaphoreType.DMA,
            pltpu.SemaphoreType.DMA,
            pltpu.SemaphoreType.DMA,
            pltpu.SemaphoreType.DMA,
        ],
        compiler_params=pltpu.CompilerParams(use_tc_tiling_on_sc=False,
                                             needs_layout_passes=False),
    )
    out5 = emb_kernel(xv, emb_table)
    return (out5.transpose(2, 4, 0, 1, 3)
            .reshape(batch, seq, WORD_DIM))
